# baseline (speedup 1.0000x reference)
import functools

import jax
import jax.numpy as jnp
from jax import lax
from jax.experimental import pallas as pl
from jax.experimental.pallas import tpu as pltpu
from jax.experimental.pallas import tpu_sc as plsc

_NC = 2
_NS = 16
_NW = _NC * _NS
_NR = 8192


def _widen_table(embeddings):
    vocab, embed = embeddings.shape
    nblk = pl.cdiv(vocab, 2 * _NR)
    t_view = embeddings.T

    def body(t_ref, w_ref):
        t = t_ref[...].T
        w_ref[:, :embed] = t[:_NR]
        w_ref[:, embed:] = t[_NR:]

    return pl.pallas_call(
        body,
        grid=(nblk,),
        in_specs=[pl.BlockSpec((embed, 2 * _NR), lambda i: (0, i))],
        out_specs=pl.BlockSpec((_NR, 2 * embed), lambda i: (i, 0)),
        out_shape=jax.ShapeDtypeStruct((nblk * _NR, 2 * embed), embeddings.dtype),
        compiler_params=pltpu.CompilerParams(
            dimension_semantics=("parallel",)
        ),
    )(t_view)


def kernel(x, embeddings):
    batch, hist = x.shape
    embed = embeddings.shape[1]
    bc = batch // _NW
    wide = _widen_table(embeddings)
    x_t = x.T

    mesh = plsc.VectorSubcoreMesh(core_axis_name="c", subcore_axis_name="s")

    @functools.partial(
        pl.kernel,
        mesh=mesh,
        out_type=jax.ShapeDtypeStruct((hist, embed, batch), embeddings.dtype),
        compiler_params=pltpu.CompilerParams(needs_layout_passes=False),
        scratch_types=[
            pltpu.VMEM((hist * bc,), jnp.int32),
            pltpu.VMEM((hist * bc,), jnp.int32),
            pltpu.VMEM((hist * bc,), jnp.int32),
            pltpu.VMEM((2, bc, 2 * embed), jnp.float32),
            pltpu.VMEM((2, embed, bc), jnp.float32),
        ]
        + [pltpu.SemaphoreType.DMA] * 5,
    )
    def sc_gather(wide_hbm, xt_hbm, out_hbm, idx_v, srow_v, hoff_v,
                  wbuf, obuf, gsem0, gsem1, ssem0, ssem1, xsem):
        gsem = (gsem0, gsem1)
        ssem = (ssem0, ssem1)
        wid = lax.axis_index("s") * _NC + lax.axis_index("c")
        b0 = wid * bc

        @pl.loop(0, hist)
        def _(h):
            pltpu.make_async_copy(
                xt_hbm.at[h, pl.ds(b0, bc)],
                idx_v.at[pl.ds(h * bc, bc)], xsem,
            ).start()

        @pl.loop(0, hist)
        def _(h):
            pltpu.make_async_copy(
                xt_hbm.at[h, pl.ds(b0, bc)],
                idx_v.at[pl.ds(h * bc, bc)], xsem,
            ).wait()

        @pl.loop(0, hist * bc, step=16)
        def _(k):
            v = idx_v[pl.ds(k, 16)]
            srow_v[pl.ds(k, 16)] = (
                lax.shift_right_logical(v, 14) * _NR + (v & (_NR - 1))
            )
            hoff_v[pl.ds(k, 16)] = (
                lax.shift_right_logical(v, 13) & 1
            ) * embed

        def gather(h, p):
            return pltpu.make_async_copy(
                wide_hbm.at[srow_v.at[pl.ds(h * bc, bc)]], wbuf.at[p], gsem[p]
            )

        def store(h, q):
            return pltpu.make_async_copy(
                obuf.at[q], out_hbm.at[h, :, pl.ds(b0, bc)], ssem[q]
            )

        def compact(h, p, q):
            @pl.loop(0, bc, step=16)
            def _(j):
                rows = j + lax.iota(jnp.int32, 16)
                hoff = hoff_v[pl.ds(h * bc + j, 16)]
                for e0 in range(0, embed, 16):
                    vals = [
                        plsc.load_gather(wbuf.at[p], [rows, hoff + (e0 + u)])
                        for u in range(16)
                    ]
                    for u in range(16):
                        obuf.at[q][e0 + u, pl.ds(j, 16)] = vals[u]

        gather(0, 0).start()
        gather(0, 0).wait()
        gather(1, 1).start()
        compact(0, 0, 0)
        store(0, 0).start()
        gather(1, 1).wait()
        gather(2, 0).start()
        compact(1, 1, 1)
        store(1, 1).start()

        @pl.loop(2, hist - 2, step=2)
        def _(h):
            gather(h, 0).wait()
            gather(h + 1, 1).start()
            store(h, 0).wait()
            compact(h, 0, 0)
            store(h, 0).start()
            gather(h + 1, 1).wait()
            gather(h + 2, 0).start()
            store(h + 1, 1).wait()
            compact(h + 1, 1, 1)
            store(h + 1, 1).start()

        gather(hist - 2, 0).wait()
        gather(hist - 1, 1).start()
        store(hist - 2, 0).wait()
        compact(hist - 2, 0, 0)
        store(hist - 2, 0).start()
        gather(hist - 1, 1).wait()
        store(hist - 1, 1).wait()
        compact(hist - 1, 1, 1)
        store(hist - 1, 1).start()
        store(hist - 2, 0).wait()
        store(hist - 1, 1).wait()

    out_t = sc_gather(wide, x_t)
    return jnp.transpose(out_t, (2, 0, 1))
